# baseline (speedup 1.0000x reference)
import jax
import jax.numpy as jnp
from jax import lax
from jax.experimental import pallas as pl
from jax.experimental.pallas import tpu as pltpu

_BLK = 512
_NBUF = 4


def _make_body(B, N, D):
    S = N // _BLK

    def _body(w_hbm, o_hbm, buf, in_sem, out_sem):
        i = pl.program_id(0)
        slot = lax.rem(i, _NBUF)
        nxt = lax.rem(i + 1, _NBUF)

        def in_copy(j, s):
            return pltpu.make_async_copy(
                w_hbm.at[pl.ds(j * _BLK, _BLK), :], buf.at[s], in_sem.at[s]
            )

        def out_copy(j, s, b):
            return pltpu.make_async_copy(
                buf.at[s], o_hbm.at[b, pl.ds(j * _BLK, _BLK), :], out_sem.at[s]
            )

        @pl.when(i == 0)
        def _():
            in_copy(0, 0).start()

        @pl.when(i >= _NBUF - 1)
        def _():
            for b in range(B):
                out_copy(i - (_NBUF - 1), nxt, b).wait()

        @pl.when(i + 1 < S)
        def _():
            in_copy(i + 1, nxt).start()

        in_copy(i, slot).wait()
        for b in range(B):
            out_copy(i, slot, b).start()

        @pl.when(i == S - 1)
        def _():
            for j in range(S - _NBUF + 1, S):
                for b in range(B):
                    out_copy(j, j % _NBUF, b).wait()

    return _body


def kernel(x, W):
    B = x.shape[0]
    N, D = W.shape
    return pl.pallas_call(
        _make_body(B, N, D),
        grid=(N // _BLK,),
        in_specs=[pl.BlockSpec(memory_space=pl.ANY)],
        out_specs=pl.BlockSpec(memory_space=pl.ANY),
        out_shape=jax.ShapeDtypeStruct((B, N, D), W.dtype),
        scratch_shapes=[
            pltpu.VMEM((_NBUF, _BLK, D), jnp.float32),
            pltpu.SemaphoreType.DMA((_NBUF,)),
            pltpu.SemaphoreType.DMA((_NBUF,)),
        ],
    )(W)


# speedup vs baseline: 1.0668x; 1.0668x over previous
import jax
import jax.numpy as jnp
from jax import lax
from jax.experimental import pallas as pl
from jax.experimental.pallas import tpu as pltpu

_BLK = 2048
_NBUF = 3


def _make_body(B, N, D):
    S = N // _BLK

    def _body(w_hbm, o_hbm, buf, in_sem, out_sem):
        i = pl.program_id(0)
        slot = lax.rem(i, _NBUF)
        nxt = lax.rem(i + 1, _NBUF)

        def in_copy(j, s):
            return pltpu.make_async_copy(
                w_hbm.at[pl.ds(j * _BLK, _BLK), :], buf.at[s], in_sem.at[s]
            )

        def out_copy(j, s, b):
            return pltpu.make_async_copy(
                buf.at[s], o_hbm.at[b, pl.ds(j * _BLK, _BLK), :], out_sem.at[s]
            )

        @pl.when(i == 0)
        def _():
            in_copy(0, 0).start()

        @pl.when(i >= _NBUF - 1)
        def _():
            for b in range(B):
                out_copy(i - (_NBUF - 1), nxt, b).wait()

        @pl.when(i + 1 < S)
        def _():
            in_copy(i + 1, nxt).start()

        in_copy(i, slot).wait()
        for b in range(B):
            out_copy(i, slot, b).start()

        @pl.when(i == S - 1)
        def _():
            for j in range(S - _NBUF + 1, S):
                for b in range(B):
                    out_copy(j, j % _NBUF, b).wait()

    return _body


def kernel(x, W):
    B = x.shape[0]
    N, D = W.shape
    return pl.pallas_call(
        _make_body(B, N, D),
        grid=(N // _BLK,),
        in_specs=[pl.BlockSpec(memory_space=pl.ANY)],
        out_specs=pl.BlockSpec(memory_space=pl.ANY),
        out_shape=jax.ShapeDtypeStruct((B, N, D), W.dtype),
        scratch_shapes=[
            pltpu.VMEM((_NBUF, _BLK, D), jnp.float32),
            pltpu.SemaphoreType.DMA((_NBUF,)),
            pltpu.SemaphoreType.DMA((_NBUF,)),
        ],
    )(W)


# speedup vs baseline: 1.0754x; 1.0081x over previous
import jax
import jax.numpy as jnp
from jax import lax
from jax.experimental import pallas as pl
from jax.experimental.pallas import tpu as pltpu

_BLK = 2048
_NBUF = 4


def _make_body(B, N, D):
    S = N // _BLK

    def _body(w_hbm, o_hbm, buf, in_sem, out_sem):
        i = pl.program_id(0)
        slot = lax.rem(i, _NBUF)
        nxt = lax.rem(i + 1, _NBUF)

        def in_copy(j, s):
            return pltpu.make_async_copy(
                w_hbm.at[pl.ds(j * _BLK, _BLK), :], buf.at[s], in_sem.at[s]
            )

        def out_copy(j, s, b):
            return pltpu.make_async_copy(
                buf.at[s], o_hbm.at[b, pl.ds(j * _BLK, _BLK), :], out_sem.at[s]
            )

        @pl.when(i == 0)
        def _():
            in_copy(0, 0).start()

        @pl.when(i >= _NBUF - 1)
        def _():
            for b in range(B):
                out_copy(i - (_NBUF - 1), nxt, b).wait()

        @pl.when(i + 1 < S)
        def _():
            in_copy(i + 1, nxt).start()

        in_copy(i, slot).wait()
        for b in range(B):
            out_copy(i, slot, b).start()

        @pl.when(i == S - 1)
        def _():
            for j in range(S - _NBUF + 1, S):
                for b in range(B):
                    out_copy(j, j % _NBUF, b).wait()

    return _body


def kernel(x, W):
    B = x.shape[0]
    N, D = W.shape
    return pl.pallas_call(
        _make_body(B, N, D),
        grid=(N // _BLK,),
        in_specs=[pl.BlockSpec(memory_space=pl.ANY)],
        out_specs=pl.BlockSpec(memory_space=pl.ANY),
        out_shape=jax.ShapeDtypeStruct((B, N, D), W.dtype),
        scratch_shapes=[
            pltpu.VMEM((_NBUF, _BLK, D), jnp.float32),
            pltpu.SemaphoreType.DMA((_NBUF,)),
            pltpu.SemaphoreType.DMA((_NBUF,)),
        ],
    )(W)
